# baseline (speedup 1.0000x reference)
import functools

import jax
import jax.numpy as jnp
from jax import lax
from jax.experimental import pallas as pl
from jax.experimental.pallas import tpu as pltpu
from jax.experimental.pallas import tpu_sc as plsc

CLIP = 64


def _build_band_kernel(table_ref, e_ref, *, S, C, D):
    e_ref[0 : S - C, :] = jnp.broadcast_to(table_ref[0:1, :], (S - C, D))
    e_ref[S - C : S - 1 + C, :] = table_ref[1 : 2 * C, :]
    e_ref[S - 1 + C :, :] = jnp.broadcast_to(table_ref[2 * C : 2 * C + 1, :], (S - C + 1, D))


def _make_sc_window_kernel(S, D, NC, NS):
    n_rows = S // (NC * NS)
    CW = 256
    n_chunks = S // CW
    K = 16
    mesh = plsc.VectorSubcoreMesh(core_axis_name="c", subcore_axis_name="s")

    @functools.partial(
        pl.kernel,
        out_type=jax.ShapeDtypeStruct((S, S, D), jnp.float32),
        mesh=mesh,
        scratch_types=[
            pltpu.VMEM((CW + n_rows, D), jnp.float32),
            pltpu.VMEM((CW + n_rows, D), jnp.float32),
            pltpu.SemaphoreType.DMA,
            pltpu.SemaphoreType.DMA,
            pltpu.SemaphoreType.DMA,
        ],
    )
    def sc_kernel(e_hbm, out_hbm, buf0, buf1, sem_in, sem_w0, sem_w1):
        cid = lax.axis_index("c")
        sid = lax.axis_index("s")

        wid = sid * NC + cid
        base = wid * n_rows
        bufs = (buf0, buf1)
        sems = (sem_w0, sem_w1)

        def stage(c, buf):
            src_lo = pl.multiple_of(S - n_rows - base + c * CW, 8)
            return pltpu.make_async_copy(
                e_hbm.at[pl.ds(src_lo, CW + n_rows), :], buf, sem_in
            )

        def mk_wait(b):
            return pltpu.make_async_copy(
                bufs[b].at[pl.ds(0, CW), :], out_hbm.at[0, pl.ds(0, CW), :], sems[b]
            )

        stage(0, buf0).start()

        for c in range(n_chunks):
            b = c % 2
            stage(c, bufs[b]).wait()

            if c + 1 < n_chunks:
                if c >= 1:
                    def drain_prev(k, _):
                        mk_wait(1 - b).wait()
                        return 0

                    lax.fori_loop(0, K, drain_prev, 0)
                stage(c + 1, bufs[1 - b]).start()

            def body(t, _):
                @pl.when(t >= K)
                def _():
                    mk_wait(b).wait()

                r = base + t
                pltpu.make_async_copy(
                    bufs[b].at[pl.ds(n_rows - 1 - t, CW), :],
                    out_hbm.at[r, pl.ds(c * CW, CW), :],
                    sems[b],
                ).start()
                return 0

            lax.fori_loop(0, n_rows, body, 0)

        def drain_last(k, _):
            mk_wait((n_chunks - 1) % 2).wait()
            return 0

        def drain_prev_last(k, _):
            mk_wait(n_chunks % 2).wait()
            return 0

        lax.fori_loop(0, K, drain_last, 0)
        lax.fori_loop(0, K, drain_prev_last, 0)

    return sc_kernel


def _rel_pos_encoding(table, S, C, D, interpret=False):
    band = pl.pallas_call(
        lambda t, e: _build_band_kernel(t, e, S=S, C=C, D=D),
        in_specs=[pl.BlockSpec(memory_space=pltpu.VMEM)],
        out_specs=pl.BlockSpec(memory_space=pltpu.VMEM),
        out_shape=jax.ShapeDtypeStruct((2 * S, D), table.dtype),
        interpret=interpret,
    )(table)
    sc_kernel = _make_sc_window_kernel(S, D, 2, 16)
    return sc_kernel(band)


def kernel(x, encoding_matrix):
    S = x.shape[1]
    D = encoding_matrix.shape[1]
    return _rel_pos_encoding(encoding_matrix, S, CLIP, D)


# speedup vs baseline: 1.0071x; 1.0071x over previous
import functools

import jax
import jax.numpy as jnp
from jax import lax
from jax.experimental import pallas as pl
from jax.experimental.pallas import tpu as pltpu
from jax.experimental.pallas import tpu_sc as plsc

CLIP = 64


def _build_band_kernel(table_ref, e_ref, *, S, C, D):
    e_ref[0 : S - C, :] = jnp.broadcast_to(table_ref[0:1, :], (S - C, D))
    e_ref[S - C : S - 1 + C, :] = table_ref[1 : 2 * C, :]
    e_ref[S - 1 + C :, :] = jnp.broadcast_to(table_ref[2 * C : 2 * C + 1, :], (S - C + 1, D))


def _make_sc_window_kernel(S, D, NC, NS):
    n_rows = S // (NC * NS)
    CW = 512
    n_chunks = S // CW
    K = 16
    mesh = plsc.VectorSubcoreMesh(core_axis_name="c", subcore_axis_name="s")

    @functools.partial(
        pl.kernel,
        out_type=jax.ShapeDtypeStruct((S, S, D), jnp.float32),
        mesh=mesh,
        scratch_types=[
            pltpu.VMEM((CW + n_rows, D), jnp.float32),
            pltpu.SemaphoreType.DMA,
            pltpu.SemaphoreType.DMA,
        ],
    )
    def sc_kernel(e_hbm, out_hbm, buf, sem_in, sem):
        cid = lax.axis_index("c")
        sid = lax.axis_index("s")

        wid = sid * NC + cid
        base = wid * n_rows

        def mk_wait():
            return pltpu.make_async_copy(
                buf.at[pl.ds(0, CW), :], out_hbm.at[0, pl.ds(0, CW), :], sem
            )

        for c in range(n_chunks):
            src_lo = pl.multiple_of(S - n_rows - base + c * CW, 8)
            pltpu.make_async_copy(
                e_hbm.at[pl.ds(src_lo, CW + n_rows), :], buf, sem_in
            ).start()
            pltpu.make_async_copy(
                e_hbm.at[pl.ds(src_lo, CW + n_rows), :], buf, sem_in
            ).wait()

            def body(t, _):
                @pl.when(t >= K)
                def _():
                    mk_wait().wait()

                r = base + t
                pltpu.make_async_copy(
                    buf.at[pl.ds(n_rows - 1 - t, CW), :],
                    out_hbm.at[r, pl.ds(c * CW, CW), :],
                    sem,
                ).start()
                return 0

            lax.fori_loop(0, n_rows, body, 0)

            def drain(k, _):
                mk_wait().wait()
                return 0

            lax.fori_loop(0, K, drain, 0)

    return sc_kernel


def _rel_pos_encoding(table, S, C, D, interpret=False):
    band = pl.pallas_call(
        lambda t, e: _build_band_kernel(t, e, S=S, C=C, D=D),
        in_specs=[pl.BlockSpec(memory_space=pltpu.VMEM)],
        out_specs=pl.BlockSpec(memory_space=pltpu.VMEM),
        out_shape=jax.ShapeDtypeStruct((2 * S, D), table.dtype),
        interpret=interpret,
    )(table)
    sc_kernel = _make_sc_window_kernel(S, D, 2, 16)
    return sc_kernel(band)


def kernel(x, encoding_matrix):
    S = x.shape[1]
    D = encoding_matrix.shape[1]
    return _rel_pos_encoding(encoding_matrix, S, CLIP, D)
